# baseline (speedup 1.0000x reference)
import functools

import jax
import jax.numpy as jnp
from jax import lax
from jax.experimental import pallas as pl
from jax.experimental.pallas import tpu as pltpu
from jax.experimental.pallas import tpu_sc as plsc

_NC = 2
_NS = 16
_NW = _NC * _NS

_B = 16384
_EMB = 32
_QUARTERS = 8
_NROWS = 1000000
_DT_COLS = 4096
_DT_GRID = 32
_GROWS = _DT_COLS * _DT_GRID
_QOFFS = (0, 122880, 249856, 372736, 499712, 622592, 749568, 872448)
_BPW = _B // _NW


@functools.cache
def _gather_wide_kernel():
    mesh = plsc.VectorSubcoreMesh(
        core_axis_name="c", subcore_axis_name="s", num_cores=_NC, num_subcores=_NS
    )

    @functools.partial(
        pl.kernel,
        out_type=(
            jax.ShapeDtypeStruct((_B, 128), jnp.float32),
            jax.ShapeDtypeStruct((_B, 128), jnp.float32),
        ),
        mesh=mesh,
        compiler_params=pltpu.CompilerParams(use_tc_tiling_on_sc=True),
        scratch_types=[
            pltpu.VMEM((_BPW,), jnp.int32),
            pltpu.VMEM((_BPW, 128), jnp.float32),
            pltpu.SemaphoreType.DMA,
        ],
    )
    def gather_wide(
        urow_hbm,
        rrow_hbm,
        gu_hbm,
        gr_hbm,
        uout_hbm,
        rout_hbm,
        idx_v,
        rows_v,
        sem,
    ):
        wid = lax.axis_index("s") * _NC + lax.axis_index("c")
        base = wid * _BPW
        pltpu.sync_copy(urow_hbm.at[pl.ds(base, _BPW)], idx_v)
        pltpu.async_copy(gu_hbm.at[idx_v], rows_v, sem).wait()
        pltpu.sync_copy(rows_v, uout_hbm.at[pl.ds(base, _BPW)])
        pltpu.sync_copy(rrow_hbm.at[pl.ds(base, _BPW)], idx_v)
        pltpu.async_copy(gr_hbm.at[idx_v], rows_v, sem).wait()
        pltpu.sync_copy(rows_v, rout_hbm.at[pl.ds(base, _BPW)])

    return gather_wide


def _mxu_t(x):
    eye = jnp.eye(128, dtype=jnp.float32)
    return lax.dot_general(
        x, eye, (((0,), (0,)), ((), ())), preferred_element_type=jnp.float32
    )


def _bf16_bits(x):
    return lax.bitcast_convert_type(
        x.astype(jnp.bfloat16).astype(jnp.float32), jnp.int32
    )


def _pack_pair(lo, hi):
    bits = _bf16_bits(hi) | lax.shift_right_logical(_bf16_bits(lo), 16)
    return lax.bitcast_convert_type(bits, jnp.float32)


def _detile_body(*refs):
    u = refs[:4]
    uh = refs[4:8]
    r = refs[8:12]
    rh = refs[12:16]
    yu_ref, yr_ref = refs[16], refs[17]
    cat = lambda blks: jnp.concatenate([b[...] for b in blks], axis=0)
    yu_ref[...] = _pack_pair(_mxu_t(cat(u)), _mxu_t(cat(uh)))
    yr_ref[...] = _pack_pair(_mxu_t(cat(r)), _mxu_t(cat(rh)))


def _detile(tab_t_u, tab_t_r):
    in_specs = [
        pl.BlockSpec((_EMB, _DT_COLS), lambda i, k=o // _DT_COLS: (0, i + k))
        for o in _QOFFS
    ]
    in_specs = in_specs + in_specs
    return pl.pallas_call(
        _detile_body,
        grid=(_DT_GRID,),
        compiler_params=pltpu.CompilerParams(fuse_transposed_lhs_in_matmul=True),
        in_specs=in_specs,
        out_specs=[
            pl.BlockSpec((_DT_COLS, 128), lambda i: (i, 0)),
            pl.BlockSpec((_DT_COLS, 128), lambda i: (i, 0)),
        ],
        out_shape=[jax.ShapeDtypeStruct((_GROWS, 128), jnp.float32)] * 2,
    )(*([tab_t_u] * _QUARTERS + [tab_t_r] * _QUARTERS))


_BM = 1024


def _unpack_select(g_ref, p_ref):
    bits = lax.bitcast_convert_type(g_ref[...], jnp.int32)
    lo = lax.bitcast_convert_type(lax.shift_left(bits, 16), jnp.float32)
    hi = lax.bitcast_convert_type(bits & jnp.int32(-65536), jnp.float32)
    lane_grp = lax.broadcasted_iota(jnp.int32, (_BM, 128), 1) >> 5
    p = p_ref[...]
    win = lane_grp == (p & 3)
    zero = jnp.zeros((), jnp.float32)
    return jnp.where(win & (p < 4), lo, zero) + jnp.where(win & (p >= 4), hi, zero)


def _mlp_body(
    gu_ref, gr_ref, pu_ref, pr_ref, w1u_ref, w1r_ref, b1_ref, w2_ref, b2_ref,
    w3_ref, b3_ref, o_ref
):
    xu = _unpack_select(gu_ref, pu_ref)
    xr = _unpack_select(gr_ref, pr_ref)
    h = jnp.dot(xu, w1u_ref[...], preferred_element_type=jnp.float32)
    h = h + jnp.dot(xr, w1r_ref[...], preferred_element_type=jnp.float32)
    h = jnp.maximum(h + b1_ref[...], 0.0)
    h = jnp.dot(h, w2_ref[...], preferred_element_type=jnp.float32) + b2_ref[...]
    h = jnp.maximum(h, 0.0)
    z = jnp.dot(h, w3_ref[...], preferred_element_type=jnp.float32) + b3_ref[...]
    o_ref[...] = 1.0 / (1.0 + jnp.exp(-z))


def _mlp(gu, gr, pu, pr, w1u, w1r, b1, w2, b2, w3, b3):
    full = lambda i: (0, 0)
    return pl.pallas_call(
        _mlp_body,
        grid=(_B // _BM,),
        in_specs=[
            pl.BlockSpec((_BM, 128), lambda i: (i, 0)),
            pl.BlockSpec((_BM, 128), lambda i: (i, 0)),
            pl.BlockSpec((_BM, 1), lambda i: (i, 0)),
            pl.BlockSpec((_BM, 1), lambda i: (i, 0)),
            pl.BlockSpec((128, 64), full),
            pl.BlockSpec((128, 64), full),
            pl.BlockSpec((1, 64), full),
            pl.BlockSpec((64, 32), full),
            pl.BlockSpec((1, 32), full),
            pl.BlockSpec((32, 1), full),
            pl.BlockSpec((1, 1), full),
        ],
        out_specs=pl.BlockSpec((_BM, 1), lambda i: (i, 0)),
        out_shape=jax.ShapeDtypeStruct((_B, 1), jnp.float32),
    )(gu, gr, pu, pr, w1u, w1r, b1, w2, b2, w3, b3)


def kernel(user, resource, user_table, res_table, W1, b1, W2, b2, W3, b3):
    gu_tab, gr_tab = _detile(user_table.T, res_table.T)
    offs = jnp.asarray(_QOFFS, dtype=jnp.int32)
    pu_full = jnp.searchsorted(offs, user, side="right").astype(jnp.int32) - 1
    pr_full = jnp.searchsorted(offs, resource, side="right").astype(jnp.int32) - 1
    urow = user - offs[pu_full]
    rrow = resource - offs[pr_full]
    gu, gr = _gather_wide_kernel()(urow, rrow, gu_tab, gr_tab)
    pu = pu_full.reshape(_B, 1)
    pr = pr_full.reshape(_B, 1)
    return _mlp(
        gu,
        gr,
        pu,
        pr,
        jnp.tile(W1[:_EMB], (4, 1)),
        jnp.tile(W1[_EMB:], (4, 1)),
        b1.reshape(1, 64),
        W2,
        b2.reshape(1, 32),
        W3,
        b3.reshape(1, 1),
    )


# speedup vs baseline: 1.0149x; 1.0149x over previous
import functools

import jax
import jax.numpy as jnp
from jax import lax
from jax.experimental import pallas as pl
from jax.experimental.pallas import tpu as pltpu
from jax.experimental.pallas import tpu_sc as plsc

_NC = 2
_NS = 16
_NW = _NC * _NS

_B = 16384
_EMB = 32
_QUARTERS = 8
_NROWS = 1000000
_DT_COLS = 2048
_DT_GRID = 62
_QOFF = _DT_COLS * (_DT_GRID - 1)
_GROWS = _DT_COLS * _DT_GRID
_BPW = _B // _NW


@functools.cache
def _gather_wide_kernel():
    mesh = plsc.VectorSubcoreMesh(
        core_axis_name="c", subcore_axis_name="s", num_cores=_NC, num_subcores=_NS
    )

    @functools.partial(
        pl.kernel,
        out_type=(
            jax.ShapeDtypeStruct((_B, 128), jnp.float32),
            jax.ShapeDtypeStruct((_B, 128), jnp.float32),
        ),
        mesh=mesh,
        compiler_params=pltpu.CompilerParams(use_tc_tiling_on_sc=True),
        scratch_types=[
            pltpu.VMEM((_BPW,), jnp.int32),
            pltpu.VMEM((_BPW, 128), jnp.float32),
            pltpu.SemaphoreType.DMA,
        ],
    )
    def gather_wide(
        urow_hbm,
        rrow_hbm,
        gu_hbm,
        gr_hbm,
        uout_hbm,
        rout_hbm,
        idx_v,
        rows_v,
        sem,
    ):
        wid = lax.axis_index("s") * _NC + lax.axis_index("c")
        base = wid * _BPW
        pltpu.sync_copy(urow_hbm.at[pl.ds(base, _BPW)], idx_v)
        pltpu.async_copy(gu_hbm.at[idx_v], rows_v, sem).wait()
        pltpu.sync_copy(rows_v, uout_hbm.at[pl.ds(base, _BPW)])
        pltpu.sync_copy(rrow_hbm.at[pl.ds(base, _BPW)], idx_v)
        pltpu.async_copy(gr_hbm.at[idx_v], rows_v, sem).wait()
        pltpu.sync_copy(rows_v, rout_hbm.at[pl.ds(base, _BPW)])

    return gather_wide


def _mxu_t(x):
    eye = jnp.eye(128, dtype=jnp.float32)
    return lax.dot_general(
        x, eye, (((0,), (0,)), ((), ())), preferred_element_type=jnp.float32
    )


def _bf16_bits(x):
    return lax.bitcast_convert_type(
        x.astype(jnp.bfloat16).astype(jnp.float32), jnp.int32
    )


def _pack_pair(lo, hi):
    bits = _bf16_bits(hi) | lax.shift_right_logical(_bf16_bits(lo), 16)
    return lax.bitcast_convert_type(bits, jnp.float32)


def _detile_body(*refs):
    u = refs[:4]
    uh = refs[4:8]
    r = refs[8:12]
    rh = refs[12:16]
    yu_ref, yr_ref = refs[16], refs[17]
    cat = lambda blks: jnp.concatenate([b[...] for b in blks], axis=0)
    yu_ref[...] = _pack_pair(_mxu_t(cat(u)), _mxu_t(cat(uh)))
    yr_ref[...] = _pack_pair(_mxu_t(cat(r)), _mxu_t(cat(rh)))


def _detile(tab_t_u, tab_t_r):
    in_specs = [
        pl.BlockSpec((_EMB, _DT_COLS), lambda i, p=p: (0, i + (_DT_GRID - 1) * p))
        for p in range(_QUARTERS)
    ]
    in_specs = in_specs + in_specs
    return pl.pallas_call(
        _detile_body,
        grid=(_DT_GRID,),
        compiler_params=pltpu.CompilerParams(fuse_transposed_lhs_in_matmul=True),
        in_specs=in_specs,
        out_specs=[
            pl.BlockSpec((_DT_COLS, 128), lambda i: (i, 0)),
            pl.BlockSpec((_DT_COLS, 128), lambda i: (i, 0)),
        ],
        out_shape=[jax.ShapeDtypeStruct((_GROWS, 128), jnp.float32)] * 2,
    )(*([tab_t_u] * _QUARTERS + [tab_t_r] * _QUARTERS))


_BM = 1024


def _unpack_select(g_ref, p_ref):
    bits = lax.bitcast_convert_type(g_ref[...], jnp.int32)
    lo = lax.bitcast_convert_type(lax.shift_left(bits, 16), jnp.float32)
    hi = lax.bitcast_convert_type(bits & jnp.int32(-65536), jnp.float32)
    lane_grp = lax.broadcasted_iota(jnp.int32, (_BM, 128), 1) >> 5
    p = p_ref[...]
    win = lane_grp == (p & 3)
    zero = jnp.zeros((), jnp.float32)
    return jnp.where(win & (p < 4), lo, zero) + jnp.where(win & (p >= 4), hi, zero)


def _mlp_body(
    gu_ref, gr_ref, pu_ref, pr_ref, w1u_ref, w1r_ref, b1_ref, w2_ref, b2_ref,
    w3_ref, b3_ref, o_ref
):
    xu = _unpack_select(gu_ref, pu_ref)
    xr = _unpack_select(gr_ref, pr_ref)
    h = jnp.dot(xu, w1u_ref[...], preferred_element_type=jnp.float32)
    h = h + jnp.dot(xr, w1r_ref[...], preferred_element_type=jnp.float32)
    h = jnp.maximum(h + b1_ref[...], 0.0)
    h = jnp.dot(h, w2_ref[...], preferred_element_type=jnp.float32) + b2_ref[...]
    h = jnp.maximum(h, 0.0)
    z = jnp.dot(h, w3_ref[...], preferred_element_type=jnp.float32) + b3_ref[...]
    o_ref[...] = 1.0 / (1.0 + jnp.exp(-z))


def _mlp(gu, gr, pu, pr, w1u, w1r, b1, w2, b2, w3, b3):
    full = lambda i: (0, 0)
    return pl.pallas_call(
        _mlp_body,
        grid=(_B // _BM,),
        in_specs=[
            pl.BlockSpec((_BM, 128), lambda i: (i, 0)),
            pl.BlockSpec((_BM, 128), lambda i: (i, 0)),
            pl.BlockSpec((_BM, 1), lambda i: (i, 0)),
            pl.BlockSpec((_BM, 1), lambda i: (i, 0)),
            pl.BlockSpec((128, 64), full),
            pl.BlockSpec((128, 64), full),
            pl.BlockSpec((1, 64), full),
            pl.BlockSpec((64, 32), full),
            pl.BlockSpec((1, 32), full),
            pl.BlockSpec((32, 1), full),
            pl.BlockSpec((1, 1), full),
        ],
        out_specs=pl.BlockSpec((_BM, 1), lambda i: (i, 0)),
        out_shape=jax.ShapeDtypeStruct((_B, 1), jnp.float32),
    )(gu, gr, pu, pr, w1u, w1r, b1, w2, b2, w3, b3)


def kernel(user, resource, user_table, res_table, W1, b1, W2, b2, W3, b3):
    gu_tab, gr_tab = _detile(user_table.T, res_table.T)
    pu_full = jnp.minimum(user // _QOFF, _QUARTERS - 1)
    pr_full = jnp.minimum(resource // _QOFF, _QUARTERS - 1)
    urow = user - pu_full * _QOFF
    rrow = resource - pr_full * _QOFF
    gu, gr = _gather_wide_kernel()(urow, rrow, gu_tab, gr_tab)
    pu = pu_full.reshape(_B, 1)
    pr = pr_full.reshape(_B, 1)
    return _mlp(
        gu,
        gr,
        pu,
        pr,
        jnp.tile(W1[:_EMB], (4, 1)),
        jnp.tile(W1[_EMB:], (4, 1)),
        b1.reshape(1, 64),
        W2,
        b2.reshape(1, 32),
        W3,
        b3.reshape(1, 1),
    )
